# baseline (speedup 1.0000x reference)
import functools

import jax
import jax.numpy as jnp
from jax import lax
from jax.experimental import pallas as pl
from jax.experimental.pallas import tpu as pltpu
from jax.experimental.pallas import tpu_sc as plsc

N = 10000
E = 320000
D = 128
H = 128
HH = H // 2
C = 16
G = 64

NC = 2
NS = 16

CHUNK = 128
EP = E // NS
NCH = 160
EPAD = NCH * CHUNK
N_T = 10240
ROWS_PER_TILE = N_T // NS

R = 512
GRID = N_T // R

_mesh = plsc.VectorSubcoreMesh(core_axis_name="c", subcore_axis_name="s")


@functools.partial(
    pl.kernel,
    out_type=jax.ShapeDtypeStruct((NC, N_T), jnp.float32),
    mesh=_mesh,
    scratch_types=[
        pltpu.VMEM((NCH, CHUNK), jnp.int32),
        pltpu.VMEM((CHUNK,), jnp.float32),
        pltpu.VMEM((ROWS_PER_TILE,), jnp.float32),
        pltpu.VMEM_SHARED((N_T,), jnp.float32),
    ],
)
def _sc_degree(dst_hbm, out_hbm, didx, ones, zbuf, acc):
    cid = lax.axis_index("c")
    sid = lax.axis_index("s")

    @pl.loop(0, ROWS_PER_TILE, step=16)
    def _(j):
        zbuf[pl.ds(j, 16)] = jnp.zeros((16,), jnp.float32)

    @pl.loop(0, CHUNK, step=16)
    def _(j):
        ones[pl.ds(j, 16)] = jnp.ones((16,), jnp.float32)

    pltpu.sync_copy(zbuf, acc.at[pl.ds(sid * ROWS_PER_TILE, ROWS_PER_TILE)])
    pltpu.sync_copy(dst_hbm.at[sid], didx)
    plsc.subcore_barrier()

    @pl.loop(0, NCH)
    def _(j):
        pltpu.sync_copy(ones, acc.at[didx.at[j]], add=True)

    plsc.subcore_barrier()
    pltpu.sync_copy(acc.at[pl.ds(sid * ROWS_PER_TILE, ROWS_PER_TILE)], zbuf)
    pltpu.sync_copy(zbuf, out_hbm.at[cid, pl.ds(sid * ROWS_PER_TILE, ROWS_PER_TILE)])


@functools.partial(
    pl.kernel,
    out_type=jax.ShapeDtypeStruct((NC, N_T, HH), jnp.float32),
    mesh=_mesh,
    scratch_types=[
        pltpu.VMEM((NCH, CHUNK), jnp.int32),
        pltpu.VMEM((NCH, CHUNK), jnp.int32),
        pltpu.VMEM((CHUNK, HH), jnp.float32),
        pltpu.VMEM((CHUNK, HH), jnp.float32),
        pltpu.VMEM_SHARED((N_T, HH), jnp.float32),
        pltpu.SemaphoreType.DMA,
        pltpu.SemaphoreType.DMA,
    ],
    compiler_params=pltpu.CompilerParams(use_tc_tiling_on_sc=False),
)
def _sc_scatter(t_hbm, src_hbm, dst_hbm, out_hbm, sidx, didx, bufa, bufb, acc,
                sema, semb):
    cid = lax.axis_index("c")
    sid = lax.axis_index("s")

    @pl.loop(0, CHUNK)
    def _(i):
        @pl.loop(0, HH, step=16)
        def _(j):
            bufa[i, pl.ds(j, 16)] = jnp.zeros((16,), jnp.float32)

    @pl.loop(0, ROWS_PER_TILE, step=CHUNK)
    def _(r):
        pltpu.sync_copy(bufa, acc.at[pl.ds(sid * ROWS_PER_TILE + r, CHUNK)])

    pltpu.sync_copy(src_hbm.at[sid], sidx)
    pltpu.sync_copy(dst_hbm.at[sid], didx)

    off = (cid * N_T).astype(jnp.int32)

    @pl.loop(0, NCH)
    def _(i):
        @pl.loop(0, CHUNK, step=16)
        def _(j):
            sidx[i, pl.ds(j, 16)] = sidx[i, pl.ds(j, 16)] + off

    plsc.subcore_barrier()

    @pl.loop(0, NCH, step=2)
    def _(i):
        da = pltpu.async_copy(t_hbm.at[sidx.at[i]], bufa, sema)
        db = pltpu.async_copy(t_hbm.at[sidx.at[i + 1]], bufb, semb)
        da.wait()
        pltpu.sync_copy(bufa, acc.at[didx.at[i]], add=True)
        db.wait()
        pltpu.sync_copy(bufb, acc.at[didx.at[i + 1]], add=True)

    plsc.subcore_barrier()

    @pl.loop(0, ROWS_PER_TILE, step=CHUNK)
    def _(r):
        row0 = sid * ROWS_PER_TILE + r
        pltpu.sync_copy(acc.at[pl.ds(row0, CHUNK)], bufa)
        pltpu.sync_copy(bufa, out_hbm.at[cid, pl.ds(row0, CHUNK)])


def _pre_body(x_ref, w_ref, dp_ref, t_ref, dinv_ref):
    deg = dp_ref[:, 0:1] + 1.0
    dinv = lax.rsqrt(deg)
    xw = jnp.dot(x_ref[...], w_ref[...], preferred_element_type=jnp.float32)
    t = xw * dinv
    t_ref[0] = t[:, :HH]
    t_ref[1] = t[:, HH:]
    dinv_ref[...] = dinv


def _tc_pre(x, w, dp):
    return pl.pallas_call(
        _pre_body,
        grid=(GRID,),
        in_specs=[
            pl.BlockSpec((R, D), lambda i: (i, 0)),
            pl.BlockSpec((D, H), lambda i: (0, 0)),
            pl.BlockSpec((R, NC), lambda i: (i, 0)),
        ],
        out_specs=[
            pl.BlockSpec((NC, R, HH), lambda i: (0, i, 0)),
            pl.BlockSpec((R, 1), lambda i: (i, 0)),
        ],
        out_shape=[
            jax.ShapeDtypeStruct((NC, N_T, HH), jnp.float32),
            jax.ShapeDtypeStruct((N_T, 1), jnp.float32),
        ],
    )(x, w, dp)


def _mid_body(p_ref, t_ref, dinv_ref, b_ref, w_ref, tn_ref):
    dinv = dinv_ref[...]
    s = jnp.concatenate([p_ref[0] + t_ref[0], p_ref[1] + t_ref[1]], axis=1)
    u = jnp.maximum(s * dinv + b_ref[...], 0.0)
    tn = jnp.dot(u, w_ref[...], preferred_element_type=jnp.float32) * dinv
    tn_ref[0] = tn[:, :HH]
    tn_ref[1] = tn[:, HH:]


def _tc_mid(p, t, dinv, b, w):
    return pl.pallas_call(
        _mid_body,
        grid=(GRID,),
        in_specs=[
            pl.BlockSpec((NC, R, HH), lambda i: (0, i, 0)),
            pl.BlockSpec((NC, R, HH), lambda i: (0, i, 0)),
            pl.BlockSpec((R, 1), lambda i: (i, 0)),
            pl.BlockSpec((1, H), lambda i: (0, 0)),
            pl.BlockSpec((H, H), lambda i: (0, 0)),
        ],
        out_specs=pl.BlockSpec((NC, R, HH), lambda i: (0, i, 0)),
        out_shape=jax.ShapeDtypeStruct((NC, N_T, HH), jnp.float32),
    )(p, t, dinv, b, w)


def _post_body(p_ref, t_ref, dinv_ref, b_ref, batch_ref,
               wm1_ref, bm1_ref, wm2_ref, bm2_ref, wm3_ref, bm3_ref,
               out_ref, pooled_acc, cnt_acc):
    i = pl.program_id(0)
    s = jnp.concatenate([p_ref[0] + t_ref[0], p_ref[1] + t_ref[1]], axis=1)
    u = jnp.maximum(s * dinv_ref[...] + b_ref[...], 0.0)
    gid = batch_ref[...]
    oh = (gid == lax.broadcasted_iota(jnp.int32, (R, G), 1)).astype(jnp.float32)
    part = lax.dot_general(oh, u, (((0,), (0,)), ((), ())),
                           preferred_element_type=jnp.float32)
    cpart = jnp.sum(oh, axis=0, keepdims=True)

    @pl.when(i == 0)
    def _():
        pooled_acc[...] = part
        cnt_acc[...] = cpart

    @pl.when(i > 0)
    def _():
        pooled_acc[...] += part
        cnt_acc[...] += cpart

    @pl.when(i == GRID - 1)
    def _():
        cnt = jnp.maximum(cnt_acc[...], 1.0)
        pooled = pooled_acc[...] / cnt.reshape(G, 1)
        h1 = jnp.maximum(jnp.dot(pooled, wm1_ref[...],
                                 preferred_element_type=jnp.float32)
                         + bm1_ref[...], 0.0)
        h2 = jnp.maximum(jnp.dot(h1, wm2_ref[...],
                                 preferred_element_type=jnp.float32)
                         + bm2_ref[...], 0.0)
        h3 = jnp.maximum(jnp.dot(h2, wm3_ref[...],
                                 preferred_element_type=jnp.float32)
                         + bm3_ref[...], 0.0)
        out_ref[...] = h3


def _tc_post(p, t, dinv, b, batch2d, wm1, bm1, wm2, bm2, wm3, bm3):
    return pl.pallas_call(
        _post_body,
        grid=(GRID,),
        in_specs=[
            pl.BlockSpec((NC, R, HH), lambda i: (0, i, 0)),
            pl.BlockSpec((NC, R, HH), lambda i: (0, i, 0)),
            pl.BlockSpec((R, 1), lambda i: (i, 0)),
            pl.BlockSpec((1, H), lambda i: (0, 0)),
            pl.BlockSpec((R, 1), lambda i: (i, 0)),
            pl.BlockSpec((H, 64), lambda i: (0, 0)),
            pl.BlockSpec((1, 64), lambda i: (0, 0)),
            pl.BlockSpec((64, 32), lambda i: (0, 0)),
            pl.BlockSpec((1, 32), lambda i: (0, 0)),
            pl.BlockSpec((32, C), lambda i: (0, 0)),
            pl.BlockSpec((1, C), lambda i: (0, 0)),
        ],
        out_specs=pl.BlockSpec((G, C), lambda i: (0, 0)),
        out_shape=jax.ShapeDtypeStruct((G, C), jnp.float32),
        scratch_shapes=[
            pltpu.VMEM((G, H), jnp.float32),
            pltpu.VMEM((1, G), jnp.float32),
        ],
    )(p, t, dinv, b, batch2d, wm1, bm1, wm2, bm2, wm3, bm3)


def kernel(x, edge_index, batch, W1, b1, W2, b2, Wm1, bm1, Wm2, bm2, Wm3, bm3):
    src = edge_index[0].reshape(NS, EP)
    dst = edge_index[1].reshape(NS, EP)
    srcp = jnp.pad(src, ((0, 0), (0, EPAD - EP))).reshape(NS, NCH, CHUNK)
    pad_dst = N + (jnp.arange(EPAD - EP, dtype=jnp.int32) % (N_T - N))
    dstp = jnp.concatenate(
        [dst, jnp.broadcast_to(pad_dst, (NS, EPAD - EP))],
        axis=1).reshape(NS, NCH, CHUNK)

    xp = jnp.pad(x, ((0, N_T - N), (0, 0)))
    batch2d = jnp.pad(batch, (0, N_T - N), constant_values=G).reshape(N_T, 1)
    b1r = b1.reshape(1, H)
    b2r = b2.reshape(1, H)

    dp = _sc_degree(dstp)
    t, dinv = _tc_pre(xp, W1, jnp.transpose(dp))

    def flat(t2):
        return t2.reshape(NC * N_T, HH)

    p = _sc_scatter(flat(t), srcp, dstp)
    t = _tc_mid(p, t, dinv, b1r, W2)
    p = _sc_scatter(flat(t), srcp, dstp)
    t = _tc_mid(p, t, dinv, b2r, W2)
    p = _sc_scatter(flat(t), srcp, dstp)
    t = _tc_mid(p, t, dinv, b2r, W2)
    p = _sc_scatter(flat(t), srcp, dstp)

    return _tc_post(p, t, dinv, b2r, batch2d,
                    Wm1, bm1.reshape(1, 64), Wm2, bm2.reshape(1, 32),
                    Wm3, bm3.reshape(1, C))


# speedup vs baseline: 1.5699x; 1.5699x over previous
import functools

import jax
import jax.numpy as jnp
from jax import lax
from jax.experimental import pallas as pl
from jax.experimental.pallas import tpu as pltpu
from jax.experimental.pallas import tpu_sc as plsc

N = 10000
E = 320000
D = 128
H = 128
HH = H // 2
C = 16
G = 64

NC = 2
NS = 16

CHUNK = 128
EP = E // NS
NCH = 158
EPAD = NCH * CHUNK
N_T = 10240
ROWS_PER_TILE = N_T // NS

R = 512
GRID = N_T // R

_mesh = plsc.VectorSubcoreMesh(core_axis_name="c", subcore_axis_name="s")


@functools.partial(
    pl.kernel,
    out_type=jax.ShapeDtypeStruct((NC, N_T), jnp.float32),
    mesh=_mesh,
    scratch_types=[
        pltpu.VMEM((NCH, CHUNK), jnp.int32),
        pltpu.VMEM((CHUNK,), jnp.float32),
        pltpu.VMEM((ROWS_PER_TILE,), jnp.float32),
        pltpu.VMEM_SHARED((N_T,), jnp.float32),
    ],
)
def _sc_degree(dst_hbm, out_hbm, didx, ones, zbuf, acc):
    cid = lax.axis_index("c")
    sid = lax.axis_index("s")

    @pl.loop(0, ROWS_PER_TILE, step=16)
    def _(j):
        zbuf[pl.ds(j, 16)] = jnp.zeros((16,), jnp.float32)

    @pl.loop(0, CHUNK, step=16)
    def _(j):
        ones[pl.ds(j, 16)] = jnp.ones((16,), jnp.float32)

    pltpu.sync_copy(zbuf, acc.at[pl.ds(sid * ROWS_PER_TILE, ROWS_PER_TILE)])
    pltpu.sync_copy(dst_hbm.at[sid], didx)
    plsc.subcore_barrier()

    @pl.loop(0, NCH)
    def _(j):
        pltpu.sync_copy(ones, acc.at[didx.at[j]], add=True)

    plsc.subcore_barrier()
    pltpu.sync_copy(acc.at[pl.ds(sid * ROWS_PER_TILE, ROWS_PER_TILE)], zbuf)
    pltpu.sync_copy(zbuf, out_hbm.at[cid, pl.ds(sid * ROWS_PER_TILE, ROWS_PER_TILE)])


@functools.partial(
    pl.kernel,
    out_type=jax.ShapeDtypeStruct((NC, N_T, HH), jnp.float32),
    mesh=_mesh,
    scratch_types=[
        pltpu.VMEM((NCH, CHUNK), jnp.int32),
        pltpu.VMEM((NCH, CHUNK), jnp.int32),
        pltpu.VMEM((CHUNK, HH), jnp.float32),
        pltpu.VMEM((CHUNK, HH), jnp.float32),
        pltpu.VMEM_SHARED((N_T, HH), jnp.float32),
        pltpu.SemaphoreType.DMA,
        pltpu.SemaphoreType.DMA,
    ],
    compiler_params=pltpu.CompilerParams(use_tc_tiling_on_sc=False),
)
def _sc_scatter(t_hbm, src_hbm, dst_hbm, out_hbm, sidx, didx, bufa, bufb, acc,
                sema, semb):
    cid = lax.axis_index("c")
    sid = lax.axis_index("s")

    @pl.loop(0, CHUNK)
    def _(i):
        @pl.loop(0, HH, step=16)
        def _(j):
            bufa[i, pl.ds(j, 16)] = jnp.zeros((16,), jnp.float32)

    @pl.loop(0, ROWS_PER_TILE, step=CHUNK)
    def _(r):
        pltpu.sync_copy(bufa, acc.at[pl.ds(sid * ROWS_PER_TILE + r, CHUNK)])

    pltpu.sync_copy(src_hbm.at[sid], sidx)
    pltpu.sync_copy(dst_hbm.at[sid], didx)

    off = (cid * N_T).astype(jnp.int32)

    @pl.loop(0, NCH)
    def _(i):
        @pl.loop(0, CHUNK, step=16)
        def _(j):
            sidx[i, pl.ds(j, 16)] = sidx[i, pl.ds(j, 16)] + off

    plsc.subcore_barrier()

    @pl.loop(0, NCH, step=2)
    def _(i):
        da = pltpu.async_copy(t_hbm.at[sidx.at[i]], bufa, sema)
        db = pltpu.async_copy(t_hbm.at[sidx.at[i + 1]], bufb, semb)
        da.wait()
        pltpu.sync_copy(bufa, acc.at[didx.at[i]], add=True)
        db.wait()
        pltpu.sync_copy(bufb, acc.at[didx.at[i + 1]], add=True)

    plsc.subcore_barrier()

    @pl.loop(0, ROWS_PER_TILE, step=CHUNK)
    def _(r):
        row0 = sid * ROWS_PER_TILE + r
        pltpu.sync_copy(acc.at[pl.ds(row0, CHUNK)], bufa)
        pltpu.sync_copy(bufa, out_hbm.at[cid, pl.ds(row0, CHUNK)])


def _pre_body(x_ref, w_ref, dp_ref, t_ref, dinv_ref):
    deg = dp_ref[:, 0:1] + 1.0
    dinv = lax.rsqrt(deg)
    xw = jnp.dot(x_ref[...], w_ref[...], preferred_element_type=jnp.float32)
    t = xw * dinv
    t_ref[0] = t[:, :HH]
    t_ref[1] = t[:, HH:]
    dinv_ref[...] = dinv


def _tc_pre(x, w, dp):
    return pl.pallas_call(
        _pre_body,
        grid=(GRID,),
        in_specs=[
            pl.BlockSpec((R, D), lambda i: (i, 0)),
            pl.BlockSpec((D, H), lambda i: (0, 0)),
            pl.BlockSpec((R, NC), lambda i: (i, 0)),
        ],
        out_specs=[
            pl.BlockSpec((NC, R, HH), lambda i: (0, i, 0)),
            pl.BlockSpec((R, 1), lambda i: (i, 0)),
        ],
        out_shape=[
            jax.ShapeDtypeStruct((NC, N_T, HH), jnp.float32),
            jax.ShapeDtypeStruct((N_T, 1), jnp.float32),
        ],
    )(x, w, dp)


def _mid_body(p_ref, t_ref, dinv_ref, b_ref, w_ref, tn_ref):
    dinv = dinv_ref[...]
    s = jnp.concatenate([p_ref[0] + t_ref[0], p_ref[1] + t_ref[1]], axis=1)
    u = jnp.maximum(s * dinv + b_ref[...], 0.0)
    tn = jnp.dot(u, w_ref[...], preferred_element_type=jnp.float32) * dinv
    tn_ref[0] = tn[:, :HH]
    tn_ref[1] = tn[:, HH:]


def _tc_mid(p, t, dinv, b, w):
    return pl.pallas_call(
        _mid_body,
        grid=(GRID,),
        in_specs=[
            pl.BlockSpec((NC, R, HH), lambda i: (0, i, 0)),
            pl.BlockSpec((NC, R, HH), lambda i: (0, i, 0)),
            pl.BlockSpec((R, 1), lambda i: (i, 0)),
            pl.BlockSpec((1, H), lambda i: (0, 0)),
            pl.BlockSpec((H, H), lambda i: (0, 0)),
        ],
        out_specs=pl.BlockSpec((NC, R, HH), lambda i: (0, i, 0)),
        out_shape=jax.ShapeDtypeStruct((NC, N_T, HH), jnp.float32),
    )(p, t, dinv, b, w)


def _post_body(p_ref, t_ref, dinv_ref, b_ref, batch_ref,
               wm1_ref, bm1_ref, wm2_ref, bm2_ref, wm3_ref, bm3_ref,
               out_ref, pooled_acc, cnt_acc):
    i = pl.program_id(0)
    s = jnp.concatenate([p_ref[0] + t_ref[0], p_ref[1] + t_ref[1]], axis=1)
    u = jnp.maximum(s * dinv_ref[...] + b_ref[...], 0.0)
    gid = batch_ref[...]
    oh = (gid == lax.broadcasted_iota(jnp.int32, (R, G), 1)).astype(jnp.float32)
    part = lax.dot_general(oh, u, (((0,), (0,)), ((), ())),
                           preferred_element_type=jnp.float32)
    cpart = jnp.sum(oh, axis=0, keepdims=True)

    @pl.when(i == 0)
    def _():
        pooled_acc[...] = part
        cnt_acc[...] = cpart

    @pl.when(i > 0)
    def _():
        pooled_acc[...] += part
        cnt_acc[...] += cpart

    @pl.when(i == GRID - 1)
    def _():
        cnt = jnp.maximum(cnt_acc[...], 1.0)
        pooled = pooled_acc[...] / cnt.reshape(G, 1)
        h1 = jnp.maximum(jnp.dot(pooled, wm1_ref[...],
                                 preferred_element_type=jnp.float32)
                         + bm1_ref[...], 0.0)
        h2 = jnp.maximum(jnp.dot(h1, wm2_ref[...],
                                 preferred_element_type=jnp.float32)
                         + bm2_ref[...], 0.0)
        h3 = jnp.maximum(jnp.dot(h2, wm3_ref[...],
                                 preferred_element_type=jnp.float32)
                         + bm3_ref[...], 0.0)
        out_ref[...] = h3


def _tc_post(p, t, dinv, b, batch2d, wm1, bm1, wm2, bm2, wm3, bm3):
    return pl.pallas_call(
        _post_body,
        grid=(GRID,),
        in_specs=[
            pl.BlockSpec((NC, R, HH), lambda i: (0, i, 0)),
            pl.BlockSpec((NC, R, HH), lambda i: (0, i, 0)),
            pl.BlockSpec((R, 1), lambda i: (i, 0)),
            pl.BlockSpec((1, H), lambda i: (0, 0)),
            pl.BlockSpec((R, 1), lambda i: (i, 0)),
            pl.BlockSpec((H, 64), lambda i: (0, 0)),
            pl.BlockSpec((1, 64), lambda i: (0, 0)),
            pl.BlockSpec((64, 32), lambda i: (0, 0)),
            pl.BlockSpec((1, 32), lambda i: (0, 0)),
            pl.BlockSpec((32, C), lambda i: (0, 0)),
            pl.BlockSpec((1, C), lambda i: (0, 0)),
        ],
        out_specs=pl.BlockSpec((G, C), lambda i: (0, 0)),
        out_shape=jax.ShapeDtypeStruct((G, C), jnp.float32),
        scratch_shapes=[
            pltpu.VMEM((G, H), jnp.float32),
            pltpu.VMEM((1, G), jnp.float32),
        ],
    )(p, t, dinv, b, batch2d, wm1, bm1, wm2, bm2, wm3, bm3)


def kernel(x, edge_index, batch, W1, b1, W2, b2, Wm1, bm1, Wm2, bm2, Wm3, bm3):
    src = edge_index[0].reshape(NS, EP)
    dst = edge_index[1].reshape(NS, EP)
    srcp = jnp.pad(src, ((0, 0), (0, EPAD - EP))).reshape(NS, NCH, CHUNK)
    pad_dst = N + (jnp.arange(EPAD - EP, dtype=jnp.int32) % (N_T - N))
    dstp = jnp.concatenate(
        [dst, jnp.broadcast_to(pad_dst, (NS, EPAD - EP))],
        axis=1).reshape(NS, NCH, CHUNK)

    xp = jnp.pad(x, ((0, N_T - N), (0, 0)))
    batch2d = jnp.pad(batch, (0, N_T - N), constant_values=G).reshape(N_T, 1)
    b1r = b1.reshape(1, H)
    b2r = b2.reshape(1, H)

    dp = _sc_degree(dstp)
    t, dinv = _tc_pre(xp, W1, jnp.transpose(dp))

    def flat(t2):
        return t2.reshape(NC * N_T, HH)

    p = _sc_scatter(flat(t), srcp, dstp)
    t = _tc_mid(p, t, dinv, b1r, W2)
    p = _sc_scatter(flat(t), srcp, dstp)
    t = _tc_mid(p, t, dinv, b2r, W2)
    p = _sc_scatter(flat(t), srcp, dstp)
    t = _tc_mid(p, t, dinv, b2r, W2)
    p = _sc_scatter(flat(t), srcp, dstp)

    return _tc_post(p, t, dinv, b2r, batch2d,
                    Wm1, bm1.reshape(1, 64), Wm2, bm2.reshape(1, 32),
                    Wm3, bm3.reshape(1, C))
